# Initial kernel scaffold; baseline (speedup 1.0000x reference)
import jax
import jax.numpy as jnp
from jax.experimental import pallas as pl


def kernel(x, tok_table, cat_tok_table, W_enc, b_enc):
    raise NotImplementedError("write your pallas kernel here")


# speedup vs baseline: 3.6891x; 3.6891x over previous
import functools

import jax
import jax.numpy as jnp
from jax import lax
from jax.experimental import pallas as pl
from jax.experimental.pallas import tpu as pltpu
from jax.experimental.pallas import tpu_sc as plsc

_GW = 128
_PERIODS = 8
_S = 200


def _pos_kernel(w_ref, b_ref, p_ref):
    S, D = p_ref.shape
    pos = lax.broadcasted_iota(jnp.int32, (S, D), 0).astype(jnp.float32)
    col = lax.broadcasted_iota(jnp.int32, (S, D), 1)
    two_i = ((col // 2) * 2).astype(jnp.float32)
    inv_div = jnp.exp(two_i * (-jnp.log(10000.0) / D))
    ang = pos * inv_div
    enc = jnp.where(col % 2 == 0, jnp.sin(ang), jnp.cos(ang))
    p_ref[...] = (
        jnp.dot(enc, w_ref[...], preferred_element_type=jnp.float32) + b_ref[...]
    )


def _mm_tanh_kernel(g_ref, w_ref, p_ref, o_ref):
    acc = jnp.dot(g_ref[...], w_ref[...], preferred_element_type=jnp.float32)
    o_ref[...] = jnp.tanh(acc + p_ref[...])


def _sc_gather(table, idx2d):
    n = idx2d.shape[1]
    d = table.shape[1]
    mesh = plsc.VectorSubcoreMesh(core_axis_name="core", subcore_axis_name="subcore")

    @functools.partial(
        pl.kernel,
        out_type=jax.ShapeDtypeStruct((n, d), table.dtype),
        mesh=mesh,
    )
    def kern(tab_hbm, i_hbm, o_hbm):
        def body(i_vmem, o_vmem):
            pltpu.sync_copy(tab_hbm.at[i_vmem.at[0]], o_vmem)

        pltpu.emit_pipeline(
            body,
            grid=(n // _GW,),
            in_specs=[pl.BlockSpec((1, _GW), index_map=lambda i: (0, i))],
            out_specs=[pl.BlockSpec((_GW, d), index_map=lambda i: (i, 0))],
            core_axis_name=("core", "subcore"),
            dimension_semantics=(pltpu.PARALLEL,),
        )(i_hbm, o_hbm)

    return kern(table, idx2d)


def kernel(x, tok_table, cat_tok_table, W_enc, b_enc):
    del cat_tok_table
    B, S = x.shape
    V, D = tok_table.shape
    N = B * S
    idx2d = x.reshape(1, N).astype(jnp.int32)
    W_top = W_enc[:D]
    W_bot = W_enc[D:]

    P = pl.pallas_call(
        _pos_kernel,
        out_shape=jax.ShapeDtypeStruct((S, D), jnp.float32),
    )(W_bot, b_enc.reshape(1, D))

    G = _sc_gather(tok_table, idx2d)

    rows = _PERIODS * S
    P8 = jnp.tile(P, (_PERIODS, 1))
    out = pl.pallas_call(
        _mm_tanh_kernel,
        grid=(N // rows,),
        in_specs=[
            pl.BlockSpec((rows, D), lambda i: (i, 0)),
            pl.BlockSpec((D, D), lambda i: (0, 0)),
            pl.BlockSpec((rows, D), lambda i: (0, 0)),
        ],
        out_specs=pl.BlockSpec((rows, D), lambda i: (i, 0)),
        out_shape=jax.ShapeDtypeStruct((N, D), jnp.float32),
    )(G, W_top, P8)
    return out.reshape(B, S, D)
